# baseline (speedup 1.0000x reference)
import functools

import jax
import jax.numpy as jnp
from jax import lax
from jax.experimental import pallas as pl
from jax.experimental.pallas import tpu as pltpu
from jax.experimental.pallas import tpu_sc as plsc

NC = 2
NS = 16
NW = NC * NS
CHUNK = 128
BLK = 1280


def _sc_mesh():
    return plsc.VectorSubcoreMesh(core_axis_name="c", subcore_axis_name="s",
                                  num_cores=NC, num_subcores=NS)


@functools.lru_cache(maxsize=None)
def _make_edge_pass(n_pad, width, n_edges):
    ept = n_edges // NW
    nchunk = ept // CHUNK
    rpt = n_pad // NS

    assert nchunk % 2 == 0 and nchunk >= 4

    def body(z_hbm, src_hbm, dst_hbm, zeros_hbm, out_hbm,
             idx_s0, idx_s1, idx_d0, idx_d1, rows0, rows1,
             ssem0, ssem1, dsem0, dsem1, gsem0, gsem1, acc):
        idx_s = (idx_s0, idx_s1)
        idx_d = (idx_d0, idx_d1)
        rows = (rows0, rows1)
        ssem = (ssem0, ssem1)
        dsem = (dsem0, dsem1)
        gsem = (gsem0, gsem1)
        c = lax.axis_index("c")
        s = lax.axis_index("s")
        r0 = s * rpt
        pltpu.sync_copy(zeros_hbm.at[pl.ds(r0, rpt), :], acc.at[pl.ds(r0, rpt), :])
        plsc.subcore_barrier()
        base = (c * NS + s) * ept

        def start_idx(j, b):
            off = base + j * CHUNK
            pltpu.async_copy(src_hbm.at[pl.ds(off, CHUNK)], idx_s[b], ssem[b])
            pltpu.async_copy(dst_hbm.at[pl.ds(off, CHUNK)], idx_d[b], dsem[b])

        def wait_idx(j, b):
            off = base + j * CHUNK
            pltpu.make_async_copy(src_hbm.at[pl.ds(off, CHUNK)], idx_s[b], ssem[b]).wait()
            pltpu.make_async_copy(dst_hbm.at[pl.ds(off, CHUNK)], idx_d[b], dsem[b]).wait()

        def start_gather(b):
            pltpu.async_copy(z_hbm.at[idx_s[b]], rows[b], gsem[b])

        def wait_gather(b):
            pltpu.make_async_copy(z_hbm.at[idx_s[b]], rows[b], gsem[b]).wait()

        start_idx(0, 0)
        start_idx(1, 1)
        wait_idx(0, 0)
        start_gather(0)

        def step(i, carry):
            for k in range(2):
                jj = 2 * i + k
                b = k
                nb = 1 - k
                wait_gather(b)
                wait_idx(jj + 1, nb)
                start_gather(nb)
                pltpu.sync_copy(rows[b], acc.at[idx_d[b]], add=True)
                start_idx(jj + 2, b)
            return carry

        lax.fori_loop(0, (nchunk - 2) // 2, step, 0)
        wait_gather(0)
        wait_idx(nchunk - 1, 1)
        start_gather(1)
        pltpu.sync_copy(rows[0], acc.at[idx_d[0]], add=True)
        wait_gather(1)
        pltpu.sync_copy(rows[1], acc.at[idx_d[1]], add=True)
        plsc.subcore_barrier()
        pltpu.sync_copy(acc.at[pl.ds(r0, rpt), :], out_hbm.at[c, pl.ds(r0, rpt), :])

    return pl.kernel(
        body,
        out_type=jax.ShapeDtypeStruct((NC, n_pad, width), jnp.float32),
        mesh=_sc_mesh(),
        scratch_types=[
            pltpu.VMEM((CHUNK,), jnp.int32),
            pltpu.VMEM((CHUNK,), jnp.int32),
            pltpu.VMEM((CHUNK,), jnp.int32),
            pltpu.VMEM((CHUNK,), jnp.int32),
            pltpu.VMEM((CHUNK, width), jnp.float32),
            pltpu.VMEM((CHUNK, width), jnp.float32),
            pltpu.SemaphoreType.DMA,
            pltpu.SemaphoreType.DMA,
            pltpu.SemaphoreType.DMA,
            pltpu.SemaphoreType.DMA,
            pltpu.SemaphoreType.DMA,
            pltpu.SemaphoreType.DMA,
            pltpu.VMEM_SHARED((n_pad, width), jnp.float32),
        ],
    )


@functools.lru_cache(maxsize=None)
def _make_attr_pass(n_pad, de, n_edges):
    ept = n_edges // NW
    nchunk = ept // CHUNK
    rpt = n_pad // NS

    nacc = (n_pad // 128 - 1) * 128
    rpt2 = nacc // NS

    def body(eap_hbm, dst_hbm, zeros_hbm, s_hbm,
             idx_d0, idx_d1, eb0, eb1, dsem0, dsem1, esem0, esem1, rows, acc):
        idx_d = (idx_d0, idx_d1)
        eb = (eb0, eb1)
        dsem = (dsem0, dsem1)
        esem = (esem0, esem1)
        c = lax.axis_index("c")
        s = lax.axis_index("s")
        r0 = s * rpt2
        pltpu.sync_copy(zeros_hbm.at[pl.ds(r0, rpt2), :], acc.at[pl.ds(r0, rpt2), :])
        pltpu.sync_copy(zeros_hbm.at[pl.ds(0, CHUNK), :], rows)
        plsc.subcore_barrier()
        base = (c * NS + s) * ept

        def start_in(j, b):
            off = base + j * CHUNK
            pltpu.async_copy(dst_hbm.at[pl.ds(off, CHUNK)], idx_d[b], dsem[b])
            pltpu.async_copy(eap_hbm.at[pl.ds(off, CHUNK), :], eb[b], esem[b])

        def wait_in(j, b):
            off = base + j * CHUNK
            pltpu.make_async_copy(dst_hbm.at[pl.ds(off, CHUNK)], idx_d[b], dsem[b]).wait()
            pltpu.make_async_copy(eap_hbm.at[pl.ds(off, CHUNK), :], eb[b], esem[b]).wait()

        def process(b):
            for ee in range(CHUNK):
                rows[ee, pl.ds(0, de)] = eb[b][ee, pl.ds(0, de)]
                rows[ee, pl.ds(de, de)] = eb[b][ee, pl.ds(de, de)]
            pltpu.sync_copy(rows, acc.at[idx_d[b]], add=True)

        start_in(0, 0)

        def step(i, carry):
            for k in range(2):
                jj = 2 * i + k
                wait_in(jj, k)
                start_in(jj + 1, 1 - k)
                process(k)
            return carry

        lax.fori_loop(0, (nchunk - 1) // 2, step, 0)
        if nchunk % 2 == 0:
            wait_in(nchunk - 2, 0)
            start_in(nchunk - 1, 1)
            process(0)
            wait_in(nchunk - 1, 1)
            process(1)
        else:
            wait_in(nchunk - 1, (nchunk - 1) % 2)
            process((nchunk - 1) % 2)
        plsc.subcore_barrier()
        pltpu.sync_copy(acc.at[pl.ds(r0, rpt2), :], s_hbm.at[c, pl.ds(r0, rpt2), :])

        @pl.when(s == NS - 1)
        def _():
            pltpu.sync_copy(zeros_hbm.at[pl.ds(0, n_pad - nacc), :],
                            s_hbm.at[c, pl.ds(nacc, n_pad - nacc), :])

    return pl.kernel(
        body,
        out_type=jax.ShapeDtypeStruct((NC, n_pad, 128), jnp.float32),
        mesh=_sc_mesh(),
        scratch_types=[
            pltpu.VMEM((CHUNK,), jnp.int32),
            pltpu.VMEM((CHUNK,), jnp.int32),
            pltpu.VMEM((CHUNK, 2 * de), jnp.float32),
            pltpu.VMEM((CHUNK, 2 * de), jnp.float32),
            pltpu.SemaphoreType.DMA,
            pltpu.SemaphoreType.DMA,
            pltpu.SemaphoreType.DMA,
            pltpu.SemaphoreType.DMA,
            pltpu.VMEM((CHUNK, 128), jnp.float32),
            pltpu.VMEM_SHARED((nacc, 128), jnp.float32),
        ],
    )


def _tc_pack(env, act, e_pad):
    e, de = env.shape
    rblk = 8192
    nblk = e_pad // rblk

    def body(env_ref, act_ref, o_ref):
        i = pl.program_id(0)
        row = jax.lax.broadcasted_iota(jnp.int32, (rblk, de), 0) + i * rblk
        mask = row < e
        o_ref[:, :de] = jnp.where(mask, env_ref[...], 0.0)
        o_ref[:, de:] = jnp.where(mask, act_ref[...], 0.0)

    last = (e - 1) // rblk

    def in_map(i):
        return (jnp.minimum(i, last), 0)

    return pl.pallas_call(
        body,
        grid=(nblk,),
        in_specs=[pl.BlockSpec((rblk, de), in_map),
                  pl.BlockSpec((rblk, de), in_map)],
        out_specs=pl.BlockSpec((rblk, 2 * de), lambda i: (i, 0)),
        out_shape=jax.ShapeDtypeStruct((e_pad, 2 * de), jnp.float32),
    )(env, act)


def _dot(a, b):
    return jnp.dot(a, b, preferred_element_type=jnp.float32,
                   precision=lax.Precision.HIGHEST)


def _tc_proj(x, wa, wb):
    n, d = x.shape

    def body(x_ref, wa_ref, wb_ref, z_ref, u_ref):
        xb = x_ref[...]
        z_ref[...] = _dot(xb, wa_ref[...])
        u_ref[...] = _dot(xb, wb_ref[...])

    return pl.pallas_call(
        body,
        grid=(n // BLK,),
        in_specs=[pl.BlockSpec((BLK, d), lambda i: (i, 0)),
                  pl.BlockSpec(wa.shape, lambda i: (0, 0)),
                  pl.BlockSpec(wb.shape, lambda i: (0, 0))],
        out_specs=[pl.BlockSpec((BLK, wa.shape[1]), lambda i: (i, 0)),
                   pl.BlockSpec((BLK, wb.shape[1]), lambda i: (i, 0))],
        out_shape=[jax.ShapeDtypeStruct((n, wa.shape[1]), jnp.float32),
                   jax.ShapeDtypeStruct((n, wb.shape[1]), jnp.float32)],
    )(x, wa, wb)


def _tc_combine(u, p, s2, we, b, w_list, s_off):
    n, h_w = u.shape
    de = we.shape[0]

    def body(u_ref, p_ref, s_ref, we_ref, b_ref, *o_refs):
        sl = (s_ref[0, :, s_off:s_off + de] + s_ref[1, :, s_off:s_off + de])
        h = (u_ref[...] + p_ref[0] + p_ref[1]
             + _dot(sl, we_ref[...]) + b_ref[...])
        h = jnp.maximum(h, 0.0)
        for o_ref, w_ref in zip(o_refs[len(w_list):], o_refs[:len(w_list)]):
            o_ref[...] = _dot(h, w_ref[...])

    return pl.pallas_call(
        body,
        grid=(n // BLK,),
        in_specs=[pl.BlockSpec((BLK, h_w), lambda i: (i, 0)),
                  pl.BlockSpec((NC, BLK, h_w), lambda i: (0, i, 0)),
                  pl.BlockSpec((NC, BLK, 128), lambda i: (0, i, 0)),
                  pl.BlockSpec((de, h_w), lambda i: (0, 0)),
                  pl.BlockSpec((1, h_w), lambda i: (0, 0))]
                 + [pl.BlockSpec(w.shape, lambda i: (0, 0)) for w in w_list],
        out_specs=[pl.BlockSpec((BLK, w.shape[1]), lambda i: (i, 0))
                   for w in w_list],
        out_shape=[jax.ShapeDtypeStruct((n, w.shape[1]), jnp.float32)
                   for w in w_list],
    )(u, p, s2, we, b, *w_list)


def _tc_final(zu2, p2, sa, we2p, b2p):
    n = zu2.shape[0]

    def body(zu_ref, p_ref, s_ref, we_ref, b_ref, o_ref):
        sl = s_ref[0, :, 16:32] + s_ref[1, :, 16:32]
        o_ref[...] = (zu_ref[:, 16:32] + p_ref[0, :, :16] + p_ref[1, :, :16]
                      + _dot(sl, we_ref[...]) + b_ref[...])

    return pl.pallas_call(
        body,
        grid=(n // BLK,),
        in_specs=[pl.BlockSpec((BLK, 128), lambda i: (i, 0)),
                  pl.BlockSpec((NC, BLK, 128), lambda i: (0, i, 0)),
                  pl.BlockSpec((NC, BLK, 128), lambda i: (0, i, 0)),
                  pl.BlockSpec((16, 16), lambda i: (0, 0)),
                  pl.BlockSpec((1, 16), lambda i: (0, 0))],
        out_specs=pl.BlockSpec((BLK, 16), lambda i: (i, 0)),
        out_shape=jax.ShapeDtypeStruct((n, 16), jnp.float32),
    )(zu2, p2, sa, we2p, b2p)


def kernel(x, env_edge_attr, act_edge_attr, edge_index,
           Ws0, Wn0, We0, b0, Ws1, Wn1, We1, b1, Ws2, Wn2, We2, b2):
    n, d = x.shape
    e = edge_index.shape[1]
    n_pad = ((n + BLK - 1) // BLK) * BLK
    de = env_edge_attr.shape[1]

    e_pad = ((e + 2 * NW * CHUNK - 1) // (2 * NW * CHUNK)) * (2 * NW * CHUNK)
    pad_e = e_pad - e
    nacc = (n_pad // 128 - 1) * 128
    pad_idx = (n + jnp.arange(pad_e, dtype=jnp.int32) % (nacc - n)).astype(jnp.int32)
    src = jnp.concatenate([edge_index[0], pad_idx])
    dst = jnp.concatenate([edge_index[1], pad_idx])
    eap = _tc_pack(env_edge_attr, act_edge_attr, e_pad)
    x_p = jnp.zeros((n_pad, d), jnp.float32).at[:n].set(x)
    zeros128 = jnp.zeros((n_pad, 128), jnp.float32)

    w2cat = jnp.concatenate([
        jnp.pad(Wn2, ((0, 0), (0, 16 - Wn2.shape[1]))),
        jnp.pad(Ws2, ((0, 0), (0, 96 + 16 - Ws2.shape[1]))),
    ], axis=1)
    we2p = jnp.pad(We2, ((0, 0), (0, 16 - We2.shape[1])))
    b2p = jnp.pad(b2, (0, 16 - b2.shape[0])).reshape(1, 16)

    edge_pass128 = _make_edge_pass(n_pad, 128, e_pad)
    attr_pass = _make_attr_pass(n_pad, de, e_pad)

    z0, u0 = _tc_proj(x_p, Wn0, Ws0)
    s_p = attr_pass(eap, dst, zeros128)

    p0 = edge_pass128(z0, src, dst, zeros128)
    z1, u1 = _tc_combine(u0, p0, s_p, We0, b0.reshape(1, -1), [Wn1, Ws1], 0)
    p1 = edge_pass128(z1, src, dst, zeros128)
    (zu2,) = _tc_combine(u1, p1, s_p, We1, b1.reshape(1, -1), [w2cat], de)
    p2 = edge_pass128(zu2, src, dst, zeros128)
    out16 = _tc_final(zu2, p2, s_p, we2p, b2p)
    return out16[:n, :2]


# speedup vs baseline: 1.1783x; 1.1783x over previous
import functools

import jax
import jax.numpy as jnp
from jax import lax
from jax.experimental import pallas as pl
from jax.experimental.pallas import tpu as pltpu
from jax.experimental.pallas import tpu_sc as plsc

NC = 2
NS = 16
NW = NC * NS
CHUNK = 128
BLK = 1280


def _sc_mesh():
    return plsc.VectorSubcoreMesh(core_axis_name="c", subcore_axis_name="s",
                                  num_cores=NC, num_subcores=NS)


@functools.lru_cache(maxsize=None)
def _make_edge_pass(n_pad, width, n_edges):
    ept = n_edges // NW
    nchunk = ept // CHUNK
    rpt = n_pad // NS

    assert nchunk % 2 == 0 and nchunk >= 4

    def body(z_hbm, src_hbm, dst_hbm, zeros_hbm, out_hbm,
             idx_s0, idx_s1, idx_d0, idx_d1, rows0, rows1,
             ssem0, ssem1, dsem0, dsem1, gsem0, gsem1, acc):
        idx_s = (idx_s0, idx_s1)
        idx_d = (idx_d0, idx_d1)
        rows = (rows0, rows1)
        ssem = (ssem0, ssem1)
        dsem = (dsem0, dsem1)
        gsem = (gsem0, gsem1)
        c = lax.axis_index("c")
        s = lax.axis_index("s")
        r0 = s * rpt
        pltpu.sync_copy(zeros_hbm.at[pl.ds(r0, rpt), :], acc.at[pl.ds(r0, rpt), :])
        plsc.subcore_barrier()
        base = (c * NS + s) * ept

        def start_idx(j, b):
            off = base + j * CHUNK
            pltpu.async_copy(src_hbm.at[pl.ds(off, CHUNK)], idx_s[b], ssem[b])
            pltpu.async_copy(dst_hbm.at[pl.ds(off, CHUNK)], idx_d[b], dsem[b])

        def wait_idx(j, b):
            off = base + j * CHUNK
            pltpu.make_async_copy(src_hbm.at[pl.ds(off, CHUNK)], idx_s[b], ssem[b]).wait()
            pltpu.make_async_copy(dst_hbm.at[pl.ds(off, CHUNK)], idx_d[b], dsem[b]).wait()

        def start_gather(b):
            pltpu.async_copy(z_hbm.at[idx_s[b]], rows[b], gsem[b])

        def wait_gather(b):
            pltpu.make_async_copy(z_hbm.at[idx_s[b]], rows[b], gsem[b]).wait()

        start_idx(0, 0)
        start_idx(1, 1)
        wait_idx(0, 0)
        start_gather(0)

        def step(i, carry):
            for k in range(2):
                jj = 2 * i + k
                b = k
                nb = 1 - k
                wait_gather(b)
                wait_idx(jj + 1, nb)
                start_gather(nb)
                pltpu.sync_copy(rows[b], acc.at[idx_d[b]], add=True)
                start_idx(jj + 2, b)
            return carry

        lax.fori_loop(0, (nchunk - 2) // 2, step, 0)
        wait_gather(0)
        wait_idx(nchunk - 1, 1)
        start_gather(1)
        pltpu.sync_copy(rows[0], acc.at[idx_d[0]], add=True)
        wait_gather(1)
        pltpu.sync_copy(rows[1], acc.at[idx_d[1]], add=True)
        plsc.subcore_barrier()
        pltpu.sync_copy(acc.at[pl.ds(r0, rpt), :], out_hbm.at[c, pl.ds(r0, rpt), :])

    return pl.kernel(
        body,
        out_type=jax.ShapeDtypeStruct((NC, n_pad, width), jnp.float32),
        mesh=_sc_mesh(),
        scratch_types=[
            pltpu.VMEM((CHUNK,), jnp.int32),
            pltpu.VMEM((CHUNK,), jnp.int32),
            pltpu.VMEM((CHUNK,), jnp.int32),
            pltpu.VMEM((CHUNK,), jnp.int32),
            pltpu.VMEM((CHUNK, width), jnp.float32),
            pltpu.VMEM((CHUNK, width), jnp.float32),
            pltpu.SemaphoreType.DMA,
            pltpu.SemaphoreType.DMA,
            pltpu.SemaphoreType.DMA,
            pltpu.SemaphoreType.DMA,
            pltpu.SemaphoreType.DMA,
            pltpu.SemaphoreType.DMA,
            pltpu.VMEM_SHARED((n_pad, width), jnp.float32),
        ],
    )


@functools.lru_cache(maxsize=None)
def _make_attr_pass(n_pad, de, n_edges):
    ept = n_edges // NW
    nchunk = ept // CHUNK
    rpt = n_pad // NS

    nacc = (n_pad // 128 - 1) * 128
    rpt2 = nacc // NS

    def body(eap_hbm, dst_hbm, zeros_hbm, s_hbm,
             idx_d0, idx_d1, eb0, eb1, dsem0, dsem1, esem0, esem1, rows, acc):
        idx_d = (idx_d0, idx_d1)
        eb = (eb0, eb1)
        dsem = (dsem0, dsem1)
        esem = (esem0, esem1)
        c = lax.axis_index("c")
        s = lax.axis_index("s")
        r0 = s * rpt2
        pltpu.sync_copy(zeros_hbm.at[pl.ds(r0, rpt2), :], acc.at[pl.ds(r0, rpt2), :])
        pltpu.sync_copy(zeros_hbm.at[pl.ds(0, CHUNK), :], rows)
        plsc.subcore_barrier()
        base = (c * NS + s) * ept

        def start_in(j, b):
            off = base + j * CHUNK
            pltpu.async_copy(dst_hbm.at[pl.ds(off, CHUNK)], idx_d[b], dsem[b])
            pltpu.async_copy(eap_hbm.at[pl.ds(off, CHUNK), :], eb[b], esem[b])

        def wait_in(j, b):
            off = base + j * CHUNK
            pltpu.make_async_copy(dst_hbm.at[pl.ds(off, CHUNK)], idx_d[b], dsem[b]).wait()
            pltpu.make_async_copy(eap_hbm.at[pl.ds(off, CHUNK), :], eb[b], esem[b]).wait()

        def process(b):
            for ee in range(CHUNK):
                rows[ee, pl.ds(0, de)] = eb[b][ee, pl.ds(0, de)]
                rows[ee, pl.ds(de, de)] = eb[b][ee, pl.ds(de, de)]
            pltpu.sync_copy(rows, acc.at[idx_d[b]], add=True)

        start_in(0, 0)

        def step(i, carry):
            for k in range(2):
                jj = 2 * i + k
                wait_in(jj, k)
                start_in(jj + 1, 1 - k)
                process(k)
            return carry

        lax.fori_loop(0, (nchunk - 1) // 2, step, 0)
        if nchunk % 2 == 0:
            wait_in(nchunk - 2, 0)
            start_in(nchunk - 1, 1)
            process(0)
            wait_in(nchunk - 1, 1)
            process(1)
        else:
            wait_in(nchunk - 1, (nchunk - 1) % 2)
            process((nchunk - 1) % 2)
        plsc.subcore_barrier()
        pltpu.sync_copy(acc.at[pl.ds(r0, rpt2), :], s_hbm.at[c, pl.ds(r0, rpt2), :])

        @pl.when(s == NS - 1)
        def _():
            pltpu.sync_copy(zeros_hbm.at[pl.ds(0, n_pad - nacc), :],
                            s_hbm.at[c, pl.ds(nacc, n_pad - nacc), :])

    return pl.kernel(
        body,
        out_type=jax.ShapeDtypeStruct((NC, n_pad, 128), jnp.float32),
        mesh=_sc_mesh(),
        scratch_types=[
            pltpu.VMEM((CHUNK,), jnp.int32),
            pltpu.VMEM((CHUNK,), jnp.int32),
            pltpu.VMEM((CHUNK, 2 * de), jnp.float32),
            pltpu.VMEM((CHUNK, 2 * de), jnp.float32),
            pltpu.SemaphoreType.DMA,
            pltpu.SemaphoreType.DMA,
            pltpu.SemaphoreType.DMA,
            pltpu.SemaphoreType.DMA,
            pltpu.VMEM((CHUNK, 128), jnp.float32),
            pltpu.VMEM_SHARED((nacc, 128), jnp.float32),
        ],
    )


def _dot(a, b):
    return jnp.dot(a, b, preferred_element_type=jnp.float32)


def _tc_proj(x, wa, wb):
    n, d = x.shape

    def body(x_ref, wa_ref, wb_ref, z_ref, u_ref):
        xb = x_ref[...]
        z_ref[...] = _dot(xb, wa_ref[...])
        u_ref[...] = _dot(xb, wb_ref[...])

    return pl.pallas_call(
        body,
        grid=(n // BLK,),
        in_specs=[pl.BlockSpec((BLK, d), lambda i: (i, 0)),
                  pl.BlockSpec(wa.shape, lambda i: (0, 0)),
                  pl.BlockSpec(wb.shape, lambda i: (0, 0))],
        out_specs=[pl.BlockSpec((BLK, wa.shape[1]), lambda i: (i, 0)),
                   pl.BlockSpec((BLK, wb.shape[1]), lambda i: (i, 0))],
        out_shape=[jax.ShapeDtypeStruct((n, wa.shape[1]), jnp.float32),
                   jax.ShapeDtypeStruct((n, wb.shape[1]), jnp.float32)],
    )(x, wa, wb)


def _tc_combine(u, p, s2, we, b, w_list, s_off):
    n, h_w = u.shape
    de = we.shape[0]

    def body(u_ref, p_ref, s_ref, we_ref, b_ref, *o_refs):
        sl = (s_ref[0, :, s_off:s_off + de] + s_ref[1, :, s_off:s_off + de])
        h = (u_ref[...] + p_ref[0] + p_ref[1]
             + _dot(sl, we_ref[...]) + b_ref[...])
        h = jnp.maximum(h, 0.0)
        for o_ref, w_ref in zip(o_refs[len(w_list):], o_refs[:len(w_list)]):
            o_ref[...] = _dot(h, w_ref[...])

    return pl.pallas_call(
        body,
        grid=(n // BLK,),
        in_specs=[pl.BlockSpec((BLK, h_w), lambda i: (i, 0)),
                  pl.BlockSpec((NC, BLK, h_w), lambda i: (0, i, 0)),
                  pl.BlockSpec((NC, BLK, 128), lambda i: (0, i, 0)),
                  pl.BlockSpec((de, h_w), lambda i: (0, 0)),
                  pl.BlockSpec((1, h_w), lambda i: (0, 0))]
                 + [pl.BlockSpec(w.shape, lambda i: (0, 0)) for w in w_list],
        out_specs=[pl.BlockSpec((BLK, w.shape[1]), lambda i: (i, 0))
                   for w in w_list],
        out_shape=[jax.ShapeDtypeStruct((n, w.shape[1]), jnp.float32)
                   for w in w_list],
    )(u, p, s2, we, b, *w_list)


def _tc_final(zu2, p2, sa, we2p, b2p):
    n = zu2.shape[0]

    def body(zu_ref, p_ref, s_ref, we_ref, b_ref, o_ref):
        sl = s_ref[0, :, 16:32] + s_ref[1, :, 16:32]
        o_ref[...] = (zu_ref[:, 16:32] + p_ref[0, :, :16] + p_ref[1, :, :16]
                      + _dot(sl, we_ref[...]) + b_ref[...])

    return pl.pallas_call(
        body,
        grid=(n // BLK,),
        in_specs=[pl.BlockSpec((BLK, 128), lambda i: (i, 0)),
                  pl.BlockSpec((NC, BLK, 128), lambda i: (0, i, 0)),
                  pl.BlockSpec((NC, BLK, 128), lambda i: (0, i, 0)),
                  pl.BlockSpec((16, 16), lambda i: (0, 0)),
                  pl.BlockSpec((1, 16), lambda i: (0, 0))],
        out_specs=pl.BlockSpec((BLK, 16), lambda i: (i, 0)),
        out_shape=jax.ShapeDtypeStruct((n, 16), jnp.float32),
    )(zu2, p2, sa, we2p, b2p)


def kernel(x, env_edge_attr, act_edge_attr, edge_index,
           Ws0, Wn0, We0, b0, Ws1, Wn1, We1, b1, Ws2, Wn2, We2, b2):
    n, d = x.shape
    e = edge_index.shape[1]
    n_pad = ((n + BLK - 1) // BLK) * BLK
    de = env_edge_attr.shape[1]

    e_pad = ((e + 2 * NW * CHUNK - 1) // (2 * NW * CHUNK)) * (2 * NW * CHUNK)
    pad_e = e_pad - e
    nacc = (n_pad // 128 - 1) * 128
    pad_idx = (n + jnp.arange(pad_e, dtype=jnp.int32) % (nacc - n)).astype(jnp.int32)
    src = jnp.concatenate([edge_index[0], pad_idx])
    dst = jnp.concatenate([edge_index[1], pad_idx])
    eap = jnp.pad(jnp.concatenate([env_edge_attr, act_edge_attr], axis=1),
                  ((0, pad_e), (0, 0)))
    x_p = jnp.zeros((n_pad, d), jnp.float32).at[:n].set(x)
    zeros128 = jnp.zeros((n_pad, 128), jnp.float32)

    w2cat = jnp.concatenate([
        jnp.pad(Wn2, ((0, 0), (0, 16 - Wn2.shape[1]))),
        jnp.pad(Ws2, ((0, 0), (0, 96 + 16 - Ws2.shape[1]))),
    ], axis=1)
    we2p = jnp.pad(We2, ((0, 0), (0, 16 - We2.shape[1])))
    b2p = jnp.pad(b2, (0, 16 - b2.shape[0])).reshape(1, 16)

    edge_pass128 = _make_edge_pass(n_pad, 128, e_pad)
    attr_pass = _make_attr_pass(n_pad, de, e_pad)

    z0, u0 = _tc_proj(x_p, Wn0, Ws0)
    s_p = attr_pass(eap, dst, zeros128)

    p0 = edge_pass128(z0, src, dst, zeros128)
    z1, u1 = _tc_combine(u0, p0, s_p, We0, b0.reshape(1, -1), [Wn1, Ws1], 0)
    p1 = edge_pass128(z1, src, dst, zeros128)
    (zu2,) = _tc_combine(u1, p1, s_p, We1, b1.reshape(1, -1), [w2cat], de)
    p2 = edge_pass128(zu2, src, dst, zeros128)
    out16 = _tc_final(zu2, p2, s_p, we2p, b2p)
    return out16[:n, :2]
